# baseline (speedup 1.0000x reference)
import functools

import jax
import jax.numpy as jnp
from jax import lax
from jax.experimental import pallas as pl
from jax.experimental.pallas import tpu as pltpu
from jax.experimental.pallas import tpu_sc as plsc

N = 10000
E = 320000
D = 128
H = 32
CP = 8

NW = 32
CK = 128
CHUNKS = 79
EPW = CHUNKS * CK
EPAD = NW * EPW
NP = 10240
RPT = NP // 16


def _make_edge_pass(feat):
    mesh = plsc.VectorSubcoreMesh(core_axis_name="c", subcore_axis_name="s")

    @functools.partial(
        pl.kernel,
        out_type=jax.ShapeDtypeStruct((2, NP, feat), jnp.float32),
        mesh=mesh,
        compiler_params=pltpu.CompilerParams(use_tc_tiling_on_sc=False),
        scratch_types=[
            pltpu.VMEM((CHUNKS, CK), jnp.int32),
            pltpu.VMEM((CHUNKS, CK), jnp.int32),
            [pltpu.VMEM((CK, feat), jnp.float32) for _ in range(4)],
            [pltpu.SemaphoreType.DMA for _ in range(4)],
            pltpu.VMEM_SHARED((NP, feat), jnp.float32),
        ],
    )
    def edge_pass(rows_hbm, edges_hbm, zeros_hbm, out_hbm,
                  src_v, dst_v, bufs, sems, acc):
        cid = lax.axis_index("c")
        sid = lax.axis_index("s")
        wid = sid * 2 + cid
        r0 = sid * RPT
        pltpu.sync_copy(zeros_hbm.at[pl.ds(r0, RPT)], acc.at[pl.ds(r0, RPT)])
        pltpu.sync_copy(edges_hbm.at[0, wid], src_v)
        pltpu.sync_copy(edges_hbm.at[1, wid], dst_v)
        plsc.subcore_barrier()

        NB = 4
        for b in range(NB):
            pltpu.async_copy(rows_hbm.at[src_v.at[b]], bufs[b], sems[b])

        def step(j, b):
            pltpu.make_async_copy(
                rows_hbm.at[src_v.at[j]], bufs[b], sems[b]).wait()
            pltpu.sync_copy(bufs[b], acc.at[dst_v.at[j]], add=True)

            @pl.when(j + NB < CHUNKS)
            def _():
                pltpu.async_copy(
                    rows_hbm.at[src_v.at[j + NB]], bufs[b], sems[b])

        def body(i, carry):
            for b in range(NB):
                step(i * NB + b, b)
            return carry

        lax.fori_loop(0, CHUNKS // NB, body, 0)
        for j in range(CHUNKS - CHUNKS % NB, CHUNKS):
            step(j, j % NB)
        plsc.subcore_barrier()
        pltpu.sync_copy(acc.at[pl.ds(r0, RPT)],
                        out_hbm.at[cid, pl.ds(r0, RPT)])

    return edge_pass


_edge_pass_h = _make_edge_pass(H)


def _make_layer2_pass():
    mesh = plsc.VectorSubcoreMesh(core_axis_name="c", subcore_axis_name="s")
    NBLK = RPT // 16

    @functools.partial(
        pl.kernel,
        out_type=jax.ShapeDtypeStruct((2, NP, 2), jnp.float32),
        mesh=mesh,
        compiler_params=pltpu.CompilerParams(
            use_tc_tiling_on_sc=False, needs_layout_passes=False),
        scratch_types=[
            pltpu.VMEM((CHUNKS, CK), jnp.int32),
            pltpu.VMEM((CHUNKS, CK), jnp.int32),
            pltpu.VMEM((RPT * H,), jnp.float32),
            pltpu.VMEM((RPT * H,), jnp.float32),
            pltpu.VMEM((H * 2 * 16,), jnp.float32),
            pltpu.VMEM((RPT * 2,), jnp.float32),
            pltpu.VMEM((NP * 2,), jnp.float32),
            [pltpu.VMEM((CK, 2), jnp.float32) for _ in range(2)],
            [pltpu.SemaphoreType.DMA for _ in range(2)],
            pltpu.VMEM_SHARED((NP * 2,), jnp.float32),
            pltpu.VMEM_SHARED((NP, 2), jnp.float32),
        ],
    )
    def layer2_pass(p1_hbm, edges_hbm, w2b_hbm, zeros_hbm, out_hbm,
                    src_v, dst_v, a_v, b_v, w2v, s2v, s2loc, vals, sems,
                    s2_sh, acc):
        cid = lax.axis_index("c")
        sid = lax.axis_index("s")
        wid = sid * 2 + cid
        r0 = sid * RPT
        pltpu.sync_copy(zeros_hbm.at[pl.ds(r0, RPT)], acc.at[pl.ds(r0, RPT)])
        pltpu.sync_copy(edges_hbm.at[0, wid], src_v)
        pltpu.sync_copy(edges_hbm.at[1, wid], dst_v)
        pltpu.sync_copy(p1_hbm.at[0, pl.ds(r0 * H, RPT * H)], a_v)
        pltpu.sync_copy(p1_hbm.at[1, pl.ds(r0 * H, RPT * H)], b_v)
        pltpu.sync_copy(w2b_hbm, w2v)

        lanes = lax.iota(jnp.int32, 16)
        zero16 = jnp.zeros((16,), jnp.int32)
        one16 = jnp.ones((16,), jnp.int32)
        w2vals = [[w2v[pl.ds((d * 2 + c) * 16, 16)] for c in range(2)]
                  for d in range(H)]

        def s2blk(blk, carry):
            base = blk * (16 * H) + lanes * H
            acc0 = jnp.zeros((16,), jnp.float32)
            acc1 = jnp.zeros((16,), jnp.float32)
            for d in range(H):
                idx = base + d
                col = jnp.maximum(
                    plsc.load_gather(a_v, [idx]) + plsc.load_gather(b_v, [idx]),
                    0.0)
                acc0 = acc0 + col * w2vals[d][0]
                acc1 = acc1 + col * w2vals[d][1]
            rows2 = (blk * 16 + lanes) * 2
            plsc.store_scatter(s2v, [rows2], acc0)
            plsc.store_scatter(s2v, [rows2 + 1], acc1)
            return carry

        lax.fori_loop(0, NBLK, s2blk, 0)
        pltpu.sync_copy(s2v, s2_sh.at[pl.ds(r0 * 2, RPT * 2)])
        plsc.subcore_barrier()
        pltpu.sync_copy(s2_sh, s2loc)

        def chunk_vals(j, b):
            for g in range(CK // 16):
                rows = g * 16 + lanes
                srcv2 = src_v[j, pl.ds(g * 16, 16)] * 2
                v0 = plsc.load_gather(s2loc, [srcv2])
                v1 = plsc.load_gather(s2loc, [srcv2 + 1])
                plsc.store_scatter(vals[b], [rows, zero16], v0)
                plsc.store_scatter(vals[b], [rows, one16], v1)

        NB = 2

        def step(j, b):
            @pl.when(j >= NB)
            def _():
                pltpu.make_async_copy(
                    vals[b], acc.at[dst_v.at[j - NB]], sems[b]).wait()
            chunk_vals(j, b)
            pltpu.async_copy(vals[b], acc.at[dst_v.at[j]], sems[b], add=True)

        def body(i, carry):
            for b in range(NB):
                step(i * NB + b, b)
            return carry

        lax.fori_loop(0, CHUNKS // NB, body, 0)
        for j in range(CHUNKS - CHUNKS % NB, CHUNKS):
            step(j, j % NB)
        for k in range(NB):
            j = CHUNKS - 1 - k
            pltpu.make_async_copy(
                vals[j % NB], acc.at[dst_v.at[j]], sems[j % NB]).wait()
        plsc.subcore_barrier()
        pltpu.sync_copy(acc.at[pl.ds(r0, RPT)],
                        out_hbm.at[cid, pl.ds(r0, RPT)])

    return layer2_pass


_layer2_pass = _make_layer2_pass()


def _matmul1(x, W1):
    def body(x_ref, w_ref, o_ref):
        o_ref[...] = jnp.dot(x_ref[...], w_ref[...],
                             preferred_element_type=jnp.float32)

    return pl.pallas_call(
        body,
        grid=(10,),
        in_specs=[pl.BlockSpec((N // 10, D), lambda i: (i, 0)),
                  pl.BlockSpec((D, H), lambda i: (0, 0))],
        out_specs=pl.BlockSpec((N // 10, H), lambda i: (i, 0)),
        out_shape=jax.ShapeDtypeStruct((N, H), jnp.float32),
    )(x, W1)


def _softmax2(p):
    def body(p_ref, o_ref):
        a = p_ref[0] + p_ref[1]
        m = jnp.max(a, axis=1, keepdims=True)
        e = jnp.exp(a - m)
        o_ref[...] = e / jnp.sum(e, axis=1, keepdims=True)

    return pl.pallas_call(
        body,
        grid=(10,),
        in_specs=[pl.BlockSpec((2, N // 10, 2), lambda i: (0, i, 0))],
        out_specs=pl.BlockSpec((N // 10, 2), lambda i: (i, 0)),
        out_shape=jax.ShapeDtypeStruct((N, 2), jnp.float32),
    )(p)


def kernel(x, edge_index, W1, W2):
    npad = EPAD - E
    ar = jnp.arange(npad, dtype=jnp.int32)
    pad_cols = jnp.stack([ar % N, N + ar % (NP - N)])
    edges = jnp.concatenate([edge_index, pad_cols], axis=1)
    edges = edges.reshape(2, NW, CHUNKS, CK)
    zeros_h = jnp.zeros((NP, H), jnp.float32)
    zeros_c = jnp.zeros((NP, 2), jnp.float32)
    w2b = jnp.broadcast_to(W2[:, :, None], (H, 2, 16)).reshape(-1)

    s1 = _matmul1(x, W1)
    p1 = _edge_pass_h(s1, edges, zeros_h)
    p2 = _layer2_pass(p1.reshape(2, NP * H), edges, w2b, zeros_c)
    return _softmax2(p2)


# speedup vs baseline: 1.0176x; 1.0176x over previous
import functools

import jax
import jax.numpy as jnp
from jax import lax
from jax.experimental import pallas as pl
from jax.experimental.pallas import tpu as pltpu
from jax.experimental.pallas import tpu_sc as plsc

N = 10000
E = 320000
D = 128
H = 32
CP = 8

NW = 32
CK = 128
CHUNKS = 79
EPW = CHUNKS * CK
EPAD = NW * EPW
NP = 10240
RPT = NP // 16


def _make_edge_pass(feat):
    mesh = plsc.VectorSubcoreMesh(core_axis_name="c", subcore_axis_name="s")

    @functools.partial(
        pl.kernel,
        out_type=jax.ShapeDtypeStruct((2, NP, feat), jnp.float32),
        mesh=mesh,
        compiler_params=pltpu.CompilerParams(use_tc_tiling_on_sc=False),
        scratch_types=[
            pltpu.VMEM((CHUNKS, CK), jnp.int32),
            pltpu.VMEM((CHUNKS, CK), jnp.int32),
            [pltpu.VMEM((CK, feat), jnp.float32) for _ in range(4)],
            [pltpu.SemaphoreType.DMA for _ in range(4)],
            pltpu.VMEM_SHARED((NP, feat), jnp.float32),
        ],
    )
    def edge_pass(rows_hbm, edges_hbm, zeros_hbm, out_hbm,
                  src_v, dst_v, bufs, sems, acc):
        cid = lax.axis_index("c")
        sid = lax.axis_index("s")
        wid = sid * 2 + cid
        r0 = sid * RPT
        pltpu.sync_copy(zeros_hbm.at[pl.ds(r0, RPT)], acc.at[pl.ds(r0, RPT)])
        pltpu.sync_copy(edges_hbm.at[0, wid], src_v)
        pltpu.sync_copy(edges_hbm.at[1, wid], dst_v)
        plsc.subcore_barrier()

        NB = 4
        for b in range(NB):
            pltpu.async_copy(rows_hbm.at[src_v.at[b]], bufs[b], sems[b])

        def step(j, b):
            pltpu.make_async_copy(
                rows_hbm.at[src_v.at[j]], bufs[b], sems[b]).wait()
            pltpu.sync_copy(bufs[b], acc.at[dst_v.at[j]], add=True)

            @pl.when(j + NB < CHUNKS)
            def _():
                pltpu.async_copy(
                    rows_hbm.at[src_v.at[j + NB]], bufs[b], sems[b])

        def body(i, carry):
            for b in range(NB):
                step(i * NB + b, b)
            return carry

        lax.fori_loop(0, CHUNKS // NB, body, 0)
        for j in range(CHUNKS - CHUNKS % NB, CHUNKS):
            step(j, j % NB)
        plsc.subcore_barrier()
        pltpu.sync_copy(acc.at[pl.ds(r0, RPT)],
                        out_hbm.at[cid, pl.ds(r0, RPT)])

    return edge_pass


_edge_pass_h = _make_edge_pass(H)


def _make_layer2_pass():
    mesh = plsc.VectorSubcoreMesh(core_axis_name="c", subcore_axis_name="s")
    NBLK = RPT // 16

    @functools.partial(
        pl.kernel,
        out_type=jax.ShapeDtypeStruct((2, 2, NP), jnp.float32),
        mesh=mesh,
        compiler_params=pltpu.CompilerParams(
            use_tc_tiling_on_sc=False, needs_layout_passes=False),
        scratch_types=[
            pltpu.VMEM((CHUNKS, CK), jnp.int32),
            pltpu.VMEM((CHUNKS, CK), jnp.int32),
            pltpu.VMEM((RPT * H,), jnp.float32),
            pltpu.VMEM((RPT * H,), jnp.float32),
            pltpu.VMEM((H * 2 * 16,), jnp.float32),
            [pltpu.VMEM((RPT,), jnp.float32) for _ in range(2)],
            [[pltpu.VMEM((CK,), jnp.float32) for _ in range(2)]
             for _ in range(2)],
            [[pltpu.SemaphoreType.DMA for _ in range(2)] for _ in range(2)],
            [pltpu.VMEM_SHARED((NP,), jnp.float32) for _ in range(2)],
            [pltpu.VMEM_SHARED((NP,), jnp.float32) for _ in range(2)],
        ],
    )
    def layer2_pass(p1_hbm, edges_hbm, w2b_hbm, zeros_hbm, out_hbm,
                    src_v, dst_v, a_v, b_v, w2v, s2v, vals, sems,
                    s2_sh, acc):
        cid = lax.axis_index("c")
        sid = lax.axis_index("s")
        wid = sid * 2 + cid
        r0 = sid * RPT
        for c in range(2):
            pltpu.sync_copy(zeros_hbm.at[c, pl.ds(r0, RPT)],
                            acc[c].at[pl.ds(r0, RPT)])
        pltpu.sync_copy(edges_hbm.at[0, wid], src_v)
        pltpu.sync_copy(edges_hbm.at[1, wid], dst_v)
        pltpu.sync_copy(p1_hbm.at[0, pl.ds(r0 * H, RPT * H)], a_v)
        pltpu.sync_copy(p1_hbm.at[1, pl.ds(r0 * H, RPT * H)], b_v)
        pltpu.sync_copy(w2b_hbm, w2v)

        lanes = lax.iota(jnp.int32, 16)
        zero16 = jnp.zeros((16,), jnp.int32)
        one16 = jnp.ones((16,), jnp.int32)
        w2vals = [[w2v[pl.ds((d * 2 + c) * 16, 16)] for c in range(2)]
                  for d in range(H)]

        def s2blk(blk, carry):
            base = blk * (16 * H) + lanes * H
            acc0 = jnp.zeros((16,), jnp.float32)
            acc1 = jnp.zeros((16,), jnp.float32)
            for d in range(H):
                idx = base + d
                col = jnp.maximum(
                    plsc.load_gather(a_v, [idx]) + plsc.load_gather(b_v, [idx]),
                    0.0)
                acc0 = acc0 + col * w2vals[d][0]
                acc1 = acc1 + col * w2vals[d][1]
            rows = blk * 16 + lanes
            plsc.store_scatter(s2v[0], [rows], acc0)
            plsc.store_scatter(s2v[1], [rows], acc1)
            return carry

        lax.fori_loop(0, NBLK, s2blk, 0)
        for c in range(2):
            pltpu.sync_copy(s2v[c], s2_sh[c].at[pl.ds(r0, RPT)])
        plsc.subcore_barrier()

        NB = 2
        GS, SS = sems[0], sems[1]

        def issue_gather(j, b):
            for c in range(2):
                pltpu.async_copy(s2_sh[c].at[src_v.at[j]], vals[c][b], GS[b])

        def wait_gather(j, b):
            for c in range(2):
                pltpu.make_async_copy(
                    s2_sh[c].at[src_v.at[j]], vals[c][b], GS[b]).wait()

        def issue_scatter(j, b):
            for c in range(2):
                pltpu.async_copy(vals[c][b], acc[c].at[dst_v.at[j]], SS[b],
                                 add=True)

        def wait_scatter(j, b):
            for c in range(2):
                pltpu.make_async_copy(
                    vals[c][b], acc[c].at[dst_v.at[j]], SS[b]).wait()

        issue_gather(0, 0)
        issue_gather(1, 1)

        def step(j, b):
            wait_gather(j, b)
            issue_scatter(j, b)

            @pl.when(j + NB < CHUNKS)
            def _():
                wait_scatter(j, b)
                issue_gather(j + NB, b)

        def body(i, carry):
            for b in range(NB):
                step(i * NB + b, b)
            return carry

        lax.fori_loop(0, CHUNKS // NB, body, 0)
        for j in range(CHUNKS - CHUNKS % NB, CHUNKS):
            step(j, j % NB)
        for j in range(CHUNKS - NB, CHUNKS):
            wait_scatter(j, j % NB)
        plsc.subcore_barrier()
        for c in range(2):
            pltpu.sync_copy(acc[c].at[pl.ds(r0, RPT)],
                            out_hbm.at[cid, c, pl.ds(r0, RPT)])

    return layer2_pass


_layer2_pass = _make_layer2_pass()


def _matmul1(x, W1):
    def body(x_ref, w_ref, o_ref):
        o_ref[...] = jnp.dot(x_ref[...], w_ref[...],
                             preferred_element_type=jnp.float32)

    return pl.pallas_call(
        body,
        grid=(10,),
        in_specs=[pl.BlockSpec((N // 10, D), lambda i: (i, 0)),
                  pl.BlockSpec((D, H), lambda i: (0, 0))],
        out_specs=pl.BlockSpec((N // 10, H), lambda i: (i, 0)),
        out_shape=jax.ShapeDtypeStruct((N, H), jnp.float32),
    )(x, W1)


def _softmax2(p):
    def body(p_ref, o_ref):
        a = p_ref[0] + p_ref[1]
        m = jnp.max(a, axis=0, keepdims=True)
        e = jnp.exp(a - m)
        sm = e / jnp.sum(e, axis=0, keepdims=True)
        o_ref[...] = jnp.stack([sm[0, :N], sm[1, :N]], axis=1)

    return pl.pallas_call(
        body,
        in_specs=[pl.BlockSpec((2, 2, NP), lambda: (0, 0, 0))],
        out_specs=pl.BlockSpec((N, 2), lambda: (0, 0)),
        out_shape=jax.ShapeDtypeStruct((N, 2), jnp.float32),
    )(p)


def kernel(x, edge_index, W1, W2):
    npad = EPAD - E
    ar = jnp.arange(npad, dtype=jnp.int32)
    pad_cols = jnp.stack([ar % N, N + ar % (NP - N)])
    edges = jnp.concatenate([edge_index, pad_cols], axis=1)
    edges = edges.reshape(2, NW, CHUNKS, CK)
    zeros_h = jnp.zeros((NP, H), jnp.float32)
    zeros_c = jnp.zeros((2, NP), jnp.float32)
    w2b = jnp.broadcast_to(W2[:, :, None], (H, 2, 16)).reshape(-1)

    s1 = _matmul1(x, W1)
    p1 = _edge_pass_h(s1, edges, zeros_h)
    p2 = _layer2_pass(p1.reshape(2, NP * H), edges, w2b, zeros_c)
    return _softmax2(p2)


# speedup vs baseline: 1.0382x; 1.0202x over previous
import functools

import jax
import jax.numpy as jnp
from jax import lax
from jax.experimental import pallas as pl
from jax.experimental.pallas import tpu as pltpu
from jax.experimental.pallas import tpu_sc as plsc

N = 10000
E = 320000
D = 128
H = 32
CP = 8

NW = 32
CK = 128
CHUNKS = 79
EPW = CHUNKS * CK
EPAD = NW * EPW
NP = 10240
RPT = NP // 16


def _make_edge_pass(feat):
    mesh = plsc.VectorSubcoreMesh(core_axis_name="c", subcore_axis_name="s")

    @functools.partial(
        pl.kernel,
        out_type=jax.ShapeDtypeStruct((2, NP, feat), jnp.float32),
        mesh=mesh,
        compiler_params=pltpu.CompilerParams(use_tc_tiling_on_sc=False),
        scratch_types=[
            pltpu.VMEM((CHUNKS, CK), jnp.int32),
            pltpu.VMEM((CHUNKS, CK), jnp.int32),
            [pltpu.VMEM((CK, feat), jnp.float32) for _ in range(4)],
            [pltpu.SemaphoreType.DMA for _ in range(4)],
            pltpu.VMEM_SHARED((NP, feat), jnp.float32),
        ],
    )
    def edge_pass(rows_hbm, edges_hbm, zeros_hbm, out_hbm,
                  src_v, dst_v, bufs, sems, acc):
        cid = lax.axis_index("c")
        sid = lax.axis_index("s")
        wid = sid * 2 + cid
        r0 = sid * RPT
        pltpu.sync_copy(zeros_hbm.at[pl.ds(r0, RPT)], acc.at[pl.ds(r0, RPT)])
        pltpu.sync_copy(edges_hbm.at[0, wid], src_v)
        pltpu.sync_copy(edges_hbm.at[1, wid], dst_v)
        plsc.subcore_barrier()

        NB = 4
        for b in range(NB):
            pltpu.async_copy(rows_hbm.at[src_v.at[b]], bufs[b], sems[b])

        def step(j, b):
            pltpu.make_async_copy(
                rows_hbm.at[src_v.at[j]], bufs[b], sems[b]).wait()
            pltpu.sync_copy(bufs[b], acc.at[dst_v.at[j]], add=True)

            @pl.when(j + NB < CHUNKS)
            def _():
                pltpu.async_copy(
                    rows_hbm.at[src_v.at[j + NB]], bufs[b], sems[b])

        def body(i, carry):
            for b in range(NB):
                step(i * NB + b, b)
            return carry

        lax.fori_loop(0, CHUNKS // NB, body, 0)
        for j in range(CHUNKS - CHUNKS % NB, CHUNKS):
            step(j, j % NB)
        plsc.subcore_barrier()
        pltpu.sync_copy(acc.at[pl.ds(r0, RPT)],
                        out_hbm.at[cid, pl.ds(r0, RPT)])

    return edge_pass


_edge_pass_h = _make_edge_pass(H)


def _make_layer2_pass():
    mesh = plsc.VectorSubcoreMesh(core_axis_name="c", subcore_axis_name="s")
    NBLK = RPT // 16

    @functools.partial(
        pl.kernel,
        out_type=jax.ShapeDtypeStruct((2, 2, NP), jnp.float32),
        mesh=mesh,
        compiler_params=pltpu.CompilerParams(
            use_tc_tiling_on_sc=False, needs_layout_passes=False),
        scratch_types=[
            pltpu.VMEM((CHUNKS, CK), jnp.int32),
            pltpu.VMEM((CHUNKS, CK), jnp.int32),
            pltpu.VMEM((RPT * H,), jnp.float32),
            pltpu.VMEM((RPT * H,), jnp.float32),
            pltpu.VMEM((H * 2 * 16,), jnp.float32),
            [pltpu.VMEM((RPT,), jnp.float32) for _ in range(2)],
            [[pltpu.VMEM((CK,), jnp.float32) for _ in range(4)]
             for _ in range(2)],
            [[pltpu.SemaphoreType.DMA for _ in range(4)] for _ in range(2)],
            [pltpu.VMEM_SHARED((NP,), jnp.float32) for _ in range(2)],
            [pltpu.VMEM_SHARED((NP,), jnp.float32) for _ in range(2)],
        ],
    )
    def layer2_pass(p1_hbm, edges_hbm, w2b_hbm, zeros_hbm, out_hbm,
                    src_v, dst_v, a_v, b_v, w2v, s2v, vals, sems,
                    s2_sh, acc):
        cid = lax.axis_index("c")
        sid = lax.axis_index("s")
        wid = sid * 2 + cid
        r0 = sid * RPT
        for c in range(2):
            pltpu.sync_copy(zeros_hbm.at[c, pl.ds(r0, RPT)],
                            acc[c].at[pl.ds(r0, RPT)])
        pltpu.sync_copy(edges_hbm.at[0, wid], src_v)
        pltpu.sync_copy(edges_hbm.at[1, wid], dst_v)
        pltpu.sync_copy(p1_hbm.at[0, pl.ds(r0 * H, RPT * H)], a_v)
        pltpu.sync_copy(p1_hbm.at[1, pl.ds(r0 * H, RPT * H)], b_v)
        pltpu.sync_copy(w2b_hbm, w2v)

        lanes = lax.iota(jnp.int32, 16)
        zero16 = jnp.zeros((16,), jnp.int32)
        one16 = jnp.ones((16,), jnp.int32)
        w2vals = [[w2v[pl.ds((d * 2 + c) * 16, 16)] for c in range(2)]
                  for d in range(H)]

        def s2blk(blk, carry):
            base = blk * (16 * H) + lanes * H
            acc0 = jnp.zeros((16,), jnp.float32)
            acc1 = jnp.zeros((16,), jnp.float32)
            for d in range(H):
                idx = base + d
                col = jnp.maximum(
                    plsc.load_gather(a_v, [idx]) + plsc.load_gather(b_v, [idx]),
                    0.0)
                acc0 = acc0 + col * w2vals[d][0]
                acc1 = acc1 + col * w2vals[d][1]
            rows = blk * 16 + lanes
            plsc.store_scatter(s2v[0], [rows], acc0)
            plsc.store_scatter(s2v[1], [rows], acc1)
            return carry

        lax.fori_loop(0, NBLK, s2blk, 0)
        for c in range(2):
            pltpu.sync_copy(s2v[c], s2_sh[c].at[pl.ds(r0, RPT)])
        plsc.subcore_barrier()

        NB = 4
        GS, SS = sems[0], sems[1]

        def issue_gather(j, b):
            for c in range(2):
                pltpu.async_copy(s2_sh[c].at[src_v.at[j]], vals[c][b], GS[b])

        def wait_gather(j, b):
            for c in range(2):
                pltpu.make_async_copy(
                    s2_sh[c].at[src_v.at[j]], vals[c][b], GS[b]).wait()

        def issue_scatter(j, b):
            for c in range(2):
                pltpu.async_copy(vals[c][b], acc[c].at[dst_v.at[j]], SS[b],
                                 add=True)

        def wait_scatter(j, b):
            for c in range(2):
                pltpu.make_async_copy(
                    vals[c][b], acc[c].at[dst_v.at[j]], SS[b]).wait()

        issue_gather(0, 0)
        issue_gather(1, 1)

        def step(j, b):
            wait_gather(j, b)
            issue_scatter(j, b)

            bb = (b + 2) % NB

            @pl.when(j + 2 < CHUNKS)
            def _():
                @pl.when(j >= 2)
                def _():
                    wait_scatter(j - 2, bb)
                issue_gather(j + 2, bb)

        def body(i, carry):
            for b in range(NB):
                step(i * NB + b, b)
            return carry

        lax.fori_loop(0, CHUNKS // NB, body, 0)
        for j in range(CHUNKS - CHUNKS % NB, CHUNKS):
            step(j, j % NB)
        for j in range(CHUNKS - NB, CHUNKS):
            wait_scatter(j, j % NB)
        plsc.subcore_barrier()
        for c in range(2):
            pltpu.sync_copy(acc[c].at[pl.ds(r0, RPT)],
                            out_hbm.at[cid, c, pl.ds(r0, RPT)])

    return layer2_pass


_layer2_pass = _make_layer2_pass()


def _matmul1(x, W1):
    def body(x_ref, w_ref, o_ref):
        o_ref[...] = jnp.dot(x_ref[...], w_ref[...],
                             preferred_element_type=jnp.float32)

    return pl.pallas_call(
        body,
        grid=(10,),
        in_specs=[pl.BlockSpec((N // 10, D), lambda i: (i, 0)),
                  pl.BlockSpec((D, H), lambda i: (0, 0))],
        out_specs=pl.BlockSpec((N // 10, H), lambda i: (i, 0)),
        out_shape=jax.ShapeDtypeStruct((N, H), jnp.float32),
    )(x, W1)


def _softmax2(p):
    def body(p_ref, o_ref):
        a = p_ref[0] + p_ref[1]
        m = jnp.max(a, axis=0, keepdims=True)
        e = jnp.exp(a - m)
        sm = e / jnp.sum(e, axis=0, keepdims=True)
        o_ref[...] = jnp.stack([sm[0, :N], sm[1, :N]], axis=1)

    return pl.pallas_call(
        body,
        in_specs=[pl.BlockSpec((2, 2, NP), lambda: (0, 0, 0))],
        out_specs=pl.BlockSpec((N, 2), lambda: (0, 0)),
        out_shape=jax.ShapeDtypeStruct((N, 2), jnp.float32),
    )(p)


def kernel(x, edge_index, W1, W2):
    npad = EPAD - E
    ar = jnp.arange(npad, dtype=jnp.int32)
    pad_cols = jnp.stack([ar % N, N + ar % (NP - N)])
    edges = jnp.concatenate([edge_index, pad_cols], axis=1)
    edges = edges.reshape(2, NW, CHUNKS, CK)
    zeros_h = jnp.zeros((NP, H), jnp.float32)
    zeros_c = jnp.zeros((2, NP), jnp.float32)
    w2b = jnp.broadcast_to(W2[:, :, None], (H, 2, 16)).reshape(-1)

    s1 = _matmul1(x, W1)
    p1 = _edge_pass_h(s1, edges, zeros_h)
    p2 = _layer2_pass(p1.reshape(2, NP * H), edges, w2b, zeros_c)
    return _softmax2(p2)
